# baseline (speedup 1.0000x reference)
import functools
import jax
import jax.numpy as jnp
from jax import lax
from jax.experimental import pallas as pl
from jax.experimental.pallas import tpu as pltpu
from jax.experimental.pallas import tpu_sc as plsc

N = 10000
D = 128
E = 320000
B = 100000
NCORE = 2
NTILE = 16
EDGES_PER_TILE = E // NTILE
CHUNK = 128
NCHUNK = 158
EPT = NCHUNK * CHUNK
EPAD = NTILE * EPT
NA = N + 8
RPT = 624
TAIL0 = NTILE * RPT
TAILN = N - TAIL0
BPAD = 100352
NCH_PRED = BPAD // 128
PRED_ITERS = 25

_sc_mesh = plsc.VectorSubcoreMesh(core_axis_name="c", subcore_axis_name="s")

_WB_SIZES = (128, 128, 128, 128, 112)


def _fill_rows(buf, nrows, value):
    def row(i, carry):
        v = jnp.full((16,), value, jnp.float32)
        for j in range(D // 16):
            buf[i, pl.ds(j * 16, 16)] = v
        return carry

    lax.fori_loop(0, nrows, row, 0)


def _spmem_init(shared, buf, s):
    off = 0
    for sz in _WB_SIZES:
        pltpu.sync_copy(buf.at[pl.ds(0, sz)],
                        shared.at[pl.ds(s * RPT + off, sz)])
        off += sz

    @pl.when(s == 0)
    def _():
        pltpu.sync_copy(buf.at[pl.ds(0, TAILN)],
                        shared.at[pl.ds(TAIL0, TAILN)])


def _spmem_writeback(shared, buf, out, s, c):
    off = 0
    for sz in _WB_SIZES:
        pltpu.sync_copy(shared.at[pl.ds(s * RPT + off, sz)],
                        buf.at[pl.ds(0, sz)])
        pltpu.sync_copy(buf.at[pl.ds(0, sz)],
                        out.at[c, pl.ds(s * RPT + off, sz)])
        off += sz

    @pl.when(s == 0)
    def _():
        pltpu.sync_copy(shared.at[pl.ds(TAIL0, TAILN)],
                        buf.at[pl.ds(0, TAILN)])
        pltpu.sync_copy(buf.at[pl.ds(0, TAILN)],
                        out.at[c, pl.ds(TAIL0, TAILN)])


@functools.partial(
    pl.kernel,
    out_type=jax.ShapeDtypeStruct((NCORE, N, D), jnp.float32),
    mesh=_sc_mesh,
    scratch_types=(
        pltpu.VMEM_SHARED((NA, D), jnp.float32),
        pltpu.VMEM((CHUNK,), jnp.int32),
        pltpu.VMEM((CHUNK,), jnp.int32),
        pltpu.VMEM((CHUNK,), jnp.int32),
        pltpu.VMEM((CHUNK,), jnp.int32),
        pltpu.VMEM((CHUNK, D), jnp.float32),
        pltpu.VMEM((CHUNK, D), jnp.float32),
        pltpu.SemaphoreType.DMA,
        pltpu.SemaphoreType.DMA,
    ),
)
def _sc_segsum(x2, src, dst, s_out,
               acc_sh, idx_s0, idx_s1, idx_d0, idx_d1, rows0, rows1,
               sem_i, sem_g):
    c = lax.axis_index("c")
    s = lax.axis_index("s")
    _fill_rows(rows0, CHUNK, 0.0)
    _spmem_init(acc_sh, rows0, s)
    plsc.subcore_barrier()
    ebase = c * EPAD + s * EPT
    rows_slots = (rows0, rows1)
    idx_s_slots = (idx_s0, idx_s1)
    idx_d_slots = (idx_d0, idx_d1)

    def idx_start(i, b):
        eoff = ebase + i * CHUNK
        pltpu.async_copy(src.at[pl.ds(eoff, CHUNK)], idx_s_slots[b], sem_i)
        pltpu.async_copy(dst.at[pl.ds(eoff, CHUNK)], idx_d_slots[b], sem_i)

    def idx_wait(i, b):
        eoff = ebase + i * CHUNK
        pltpu.make_async_copy(src.at[pl.ds(eoff, CHUNK)], idx_s_slots[b],
                              sem_i).wait()
        pltpu.make_async_copy(dst.at[pl.ds(eoff, CHUNK)], idx_d_slots[b],
                              sem_i).wait()

    idx_start(0, 0)
    idx_start(1, 1)
    idx_wait(0, 0)
    pltpu.async_copy(x2.at[idx_s0], rows0, sem_g)

    def body(i2, carry):
        for b in range(2):
            i = 2 * i2 + b
            nb = 1 - b

            @pl.when(i < NCHUNK - 1)
            def _():
                idx_wait(i + 1, nb)
                pltpu.async_copy(x2.at[idx_s_slots[nb]], rows_slots[nb],
                                 sem_g)

            pltpu.make_async_copy(x2.at[idx_s_slots[b]], rows_slots[b],
                                  sem_g).wait()
            pltpu.sync_copy(rows_slots[b], acc_sh.at[idx_d_slots[b]],
                            add=True)

            @pl.when(i < NCHUNK - 2)
            def _():
                idx_start(i + 2, b)
        return carry

    lax.fori_loop(0, NCHUNK // 2, body, 0)
    plsc.subcore_barrier()
    _spmem_writeback(acc_sh, rows0, s_out, s, c)


@functools.partial(
    pl.kernel,
    out_type=jax.ShapeDtypeStruct((NCORE, N, D), jnp.float32),
    mesh=_sc_mesh,
    scratch_types=(
        pltpu.VMEM_SHARED((NA, D), jnp.float32),
        pltpu.VMEM((CHUNK,), jnp.int32),
        pltpu.VMEM((CHUNK,), jnp.int32),
        pltpu.VMEM((CHUNK, D), jnp.float32),
        pltpu.VMEM((CHUNK, D), jnp.float32),
        pltpu.SemaphoreType.DMA,
    ),
)
def _sc_segcount(dst, c_out, cnt_sh, idx_d0, idx_d1, buf, ones, sem):
    c = lax.axis_index("c")
    s = lax.axis_index("s")
    _fill_rows(buf, CHUNK, 0.0)
    _fill_rows(ones, CHUNK, 1.0)
    _spmem_init(cnt_sh, buf, s)
    plsc.subcore_barrier()
    ebase = c * EPAD + s * EPT

    idx_slots = (idx_d0, idx_d1)
    pltpu.async_copy(dst.at[pl.ds(ebase, CHUNK)], idx_d0, sem)

    def body(i2, carry):
        for b in range(2):
            i = 2 * i2 + b
            eoff = ebase + i * CHUNK

            @pl.when(i < NCHUNK - 1)
            def _():
                pltpu.async_copy(dst.at[pl.ds(eoff + CHUNK, CHUNK)],
                                 idx_slots[1 - b], sem)

            pltpu.make_async_copy(dst.at[pl.ds(eoff, CHUNK)], idx_slots[b],
                                  sem).wait()
            pltpu.sync_copy(ones, cnt_sh.at[idx_slots[b]], add=True)
        return carry

    lax.fori_loop(0, NCHUNK // 2, body, 0)
    plsc.subcore_barrier()
    _spmem_writeback(cnt_sh, buf, c_out, s, c)


@functools.partial(
    pl.kernel,
    out_type=(
        jax.ShapeDtypeStruct((BPAD, D), jnp.float32),
        jax.ShapeDtypeStruct((BPAD, D), jnp.float32),
    ),
    mesh=_sc_mesh,
    scratch_types=(
        pltpu.VMEM((128,), jnp.int32),
        pltpu.VMEM((128,), jnp.int32),
        pltpu.VMEM((128, D), jnp.float32),
        pltpu.VMEM((128, D), jnp.float32),
        pltpu.SemaphoreType.DMA,
        pltpu.SemaphoreType.DMA,
    ),
)
def _sc_pair_gather(p2, i0, i1, a_out, b_out, idx_a, idx_b, rows_a, rows_b,
                    sem_i, sem_g):
    c = lax.axis_index("c")
    s = lax.axis_index("s")
    wid = s * NCORE + c

    def body(j, carry):
        ch = wid + 32 * j

        @pl.when(ch < NCH_PRED)
        def _():
            off = ch * 128
            pltpu.async_copy(i0.at[pl.ds(off, 128)], idx_a, sem_i)
            pltpu.async_copy(i1.at[pl.ds(off, 128)], idx_b, sem_i)
            pltpu.make_async_copy(i0.at[pl.ds(off, 128)], idx_a,
                                  sem_i).wait()
            pltpu.async_copy(p2.at[idx_a], rows_a, sem_g)
            pltpu.make_async_copy(i1.at[pl.ds(off, 128)], idx_b,
                                  sem_i).wait()
            pltpu.async_copy(p2.at[idx_b], rows_b, sem_g)
            pltpu.make_async_copy(p2.at[idx_a], rows_a, sem_g).wait()
            pltpu.sync_copy(rows_a, a_out.at[pl.ds(off, 128)])
            pltpu.make_async_copy(p2.at[idx_b], rows_b, sem_g).wait()
            pltpu.sync_copy(rows_b, b_out.at[pl.ds(off, 128)])

        return carry

    lax.fori_loop(0, PRED_ITERS, body, 0)


RB = 2000
NRB = N // RB


def _tc_layer1_body(s_ref, c_ref, x_ref, wl_ref, b_ref, wr_ref, out_ref):
    agg = s_ref[0] / jnp.maximum(c_ref[0], 1.0)
    h = (jnp.dot(agg, wl_ref[0], preferred_element_type=jnp.float32)
         + b_ref[0, 0][None, :]
         + jnp.dot(x_ref[...], wr_ref[0], preferred_element_type=jnp.float32))
    out_ref[...] = jnp.maximum(h, 0.0)


def _tc_layer2_body(s_ref, c_ref, x_ref, wl_ref, b_ref, wr_ref, wp_ref,
                    out_ref):
    agg = s_ref[0] / jnp.maximum(c_ref[0], 1.0)
    h = (jnp.dot(agg, wl_ref[0], preferred_element_type=jnp.float32)
         + b_ref[0, 0][None, :]
         + jnp.dot(x_ref[...], wr_ref[0], preferred_element_type=jnp.float32))
    out_ref[...] = jnp.dot(h, wp_ref[0], preferred_element_type=jnp.float32)


PB = 2048


def _tc_pred_body(a_ref, b_ref, bp1_ref, w2_ref, bp2_ref, out_ref):
    x = jnp.maximum(a_ref[...] + b_ref[...] + bp1_ref[...], 0.0)
    z = jnp.sum(x * w2_ref[...], axis=1) + bp2_ref[0, 0]
    out_ref[...] = jax.nn.sigmoid(z)


def kernel(x_investor, x_fund, edge_index_invests, edge_index_rev,
           edge_label_index,
           W1_if_l, W1_if_r, b1_if, W1_fi_l, W1_fi_r, b1_fi,
           W2_if_l, W2_if_r, b2_if, W2_fi_l, W2_fi_r, b2_fi,
           Wp1, bp1, Wp2, bp2):
    f32 = jnp.float32

    def _pad_span(a, fill):
        a = a.reshape(NTILE, EDGES_PER_TILE)
        a = jnp.pad(a, ((0, 0), (0, EPT - EDGES_PER_TILE)),
                    constant_values=fill)
        return a.reshape(-1)

    src = jnp.concatenate([_pad_span(edge_index_invests[0], 0),
                           _pad_span(edge_index_rev[0] + N, 0)])
    dst = jnp.concatenate([_pad_span(edge_index_invests[1], N),
                           _pad_span(edge_index_rev[1], N)])

    cnt = _sc_segcount(dst)

    x1cat = jnp.concatenate([x_investor, x_fund])
    s1 = _sc_segsum(x1cat, src, dst)

    w1l = jnp.stack([W1_if_l, W1_fi_l])
    w1r = jnp.stack([W1_if_r, W1_fi_r])
    bias1 = jnp.stack([b1_if, b1_fi]).reshape(NCORE, 1, D)
    h1 = pl.pallas_call(
        _tc_layer1_body,
        grid=(NCORE, NRB),
        in_specs=[
            pl.BlockSpec((1, RB, D), lambda d, r: (d, r, 0)),
            pl.BlockSpec((1, RB, D), lambda d, r: (d, r, 0)),
            pl.BlockSpec((RB, D), lambda d, r: ((1 - d) * NRB + r, 0)),
            pl.BlockSpec((1, D, D), lambda d, r: (d, 0, 0)),
            pl.BlockSpec((1, 1, D), lambda d, r: (d, 0, 0)),
            pl.BlockSpec((1, D, D), lambda d, r: (d, 0, 0)),
        ],
        out_specs=pl.BlockSpec((RB, D), lambda d, r: ((1 - d) * NRB + r, 0)),
        out_shape=jax.ShapeDtypeStruct((NCORE * N, D), f32),
    )(s1, cnt, x1cat, w1l, bias1, w1r)

    s2 = _sc_segsum(h1, src, dst)

    w2l = jnp.stack([W2_if_l, W2_fi_l])
    w2r = jnp.stack([W2_if_r, W2_fi_r])
    bias2 = jnp.stack([b2_if, b2_fi]).reshape(NCORE, 1, D)
    wp_halves = jnp.stack([Wp1[D:], Wp1[:D]])
    p2 = pl.pallas_call(
        _tc_layer2_body,
        grid=(NCORE, NRB),
        in_specs=[
            pl.BlockSpec((1, RB, D), lambda d, r: (d, r, 0)),
            pl.BlockSpec((1, RB, D), lambda d, r: (d, r, 0)),
            pl.BlockSpec((RB, D), lambda d, r: ((1 - d) * NRB + r, 0)),
            pl.BlockSpec((1, D, D), lambda d, r: (d, 0, 0)),
            pl.BlockSpec((1, 1, D), lambda d, r: (d, 0, 0)),
            pl.BlockSpec((1, D, D), lambda d, r: (d, 0, 0)),
            pl.BlockSpec((1, D, D), lambda d, r: (d, 0, 0)),
        ],
        out_specs=pl.BlockSpec((RB, D), lambda d, r: (d * NRB + r, 0)),
        out_shape=jax.ShapeDtypeStruct((NCORE * N, D), f32),
    )(s2, cnt, h1, w2l, bias2, w2r, wp_halves)

    pad = jnp.zeros((BPAD - B,), jnp.int32)
    i0 = jnp.concatenate([edge_label_index[0] + N, pad])
    i1 = jnp.concatenate([edge_label_index[1], pad])
    ag, bg = _sc_pair_gather(p2, i0, i1)

    bp1r = bp1.reshape(1, D)
    w2row = Wp2.reshape(1, D)
    bp2r = jnp.broadcast_to(bp2.reshape(1, 1), (1, D))
    z = pl.pallas_call(
        _tc_pred_body,
        grid=(BPAD // PB,),
        in_specs=[
            pl.BlockSpec((PB, D), lambda r: (r, 0)),
            pl.BlockSpec((PB, D), lambda r: (r, 0)),
            pl.BlockSpec((1, D), lambda r: (0, 0)),
            pl.BlockSpec((1, D), lambda r: (0, 0)),
            pl.BlockSpec((1, D), lambda r: (0, 0)),
        ],
        out_specs=pl.BlockSpec((PB,), lambda r: (r,)),
        out_shape=jax.ShapeDtypeStruct((BPAD,), f32),
    )(ag, bg, bp1r, w2row, bp2r)
    return z[:B].reshape(B, 1)


# speedup vs baseline: 1.5001x; 1.5001x over previous
import functools
import jax
import jax.numpy as jnp
from jax import lax
from jax.experimental import pallas as pl
from jax.experimental.pallas import tpu as pltpu
from jax.experimental.pallas import tpu_sc as plsc

N = 10000
D = 128
E = 320000
B = 100000
NCORE = 2
NTILE = 16
EDGES_PER_TILE = E // NTILE
CHUNK = 80
NCHUNK = EDGES_PER_TILE // CHUNK
EPT = EDGES_PER_TILE
EPAD = E
NA = N
RPT = 624
TAIL0 = NTILE * RPT
TAILN = N - TAIL0
BPAD = 100352
NCH_PRED = BPAD // 128
PRED_ITERS = 25

_sc_mesh = plsc.VectorSubcoreMesh(core_axis_name="c", subcore_axis_name="s")

_WB_SIZES = (80, 80, 80, 80, 80, 80, 80, 64)


def _fill_rows(buf, nrows, value):
    def row(i, carry):
        v = jnp.full((16,), value, jnp.float32)
        for j in range(D // 16):
            buf[i, pl.ds(j * 16, 16)] = v
        return carry

    lax.fori_loop(0, nrows, row, 0)


def _spmem_init(shared, buf, s):
    off = 0
    for sz in _WB_SIZES:
        pltpu.sync_copy(buf.at[pl.ds(0, sz)],
                        shared.at[pl.ds(s * RPT + off, sz)])
        off += sz

    @pl.when(s == 0)
    def _():
        pltpu.sync_copy(buf.at[pl.ds(0, TAILN)],
                        shared.at[pl.ds(TAIL0, TAILN)])


def _spmem_writeback(shared, buf, out, s, c):
    off = 0
    for sz in _WB_SIZES:
        pltpu.sync_copy(shared.at[pl.ds(s * RPT + off, sz)],
                        buf.at[pl.ds(0, sz)])
        pltpu.sync_copy(buf.at[pl.ds(0, sz)],
                        out.at[c, pl.ds(s * RPT + off, sz)])
        off += sz

    @pl.when(s == 0)
    def _():
        pltpu.sync_copy(shared.at[pl.ds(TAIL0, TAILN)],
                        buf.at[pl.ds(0, TAILN)])
        pltpu.sync_copy(buf.at[pl.ds(0, TAILN)],
                        out.at[c, pl.ds(TAIL0, TAILN)])


@functools.partial(
    pl.kernel,
    out_type=jax.ShapeDtypeStruct((NCORE, N, D), jnp.float32),
    mesh=_sc_mesh,
    scratch_types=(
        pltpu.VMEM_SHARED((NA, D), jnp.float32),
        pltpu.VMEM((CHUNK,), jnp.int32),
        pltpu.VMEM((CHUNK,), jnp.int32),
        pltpu.VMEM((CHUNK,), jnp.int32),
        pltpu.VMEM((CHUNK,), jnp.int32),
        pltpu.VMEM((CHUNK, D), jnp.float32),
        pltpu.VMEM((CHUNK, D), jnp.float32),
        pltpu.SemaphoreType.DMA,
        pltpu.SemaphoreType.DMA,
    ),
)
def _sc_segsum(x2, src, dst, s_out,
               acc_sh, idx_s0, idx_s1, idx_d0, idx_d1, rows0, rows1,
               sem_i, sem_g):
    c = lax.axis_index("c")
    s = lax.axis_index("s")
    _fill_rows(rows0, CHUNK, 0.0)
    _spmem_init(acc_sh, rows0, s)
    plsc.subcore_barrier()
    ebase = c * EPAD + s * EPT
    rows_slots = (rows0, rows1)
    idx_s_slots = (idx_s0, idx_s1)
    idx_d_slots = (idx_d0, idx_d1)

    def idx_start(i, b):
        eoff = ebase + i * CHUNK
        pltpu.async_copy(src.at[pl.ds(eoff, CHUNK)], idx_s_slots[b], sem_i)
        pltpu.async_copy(dst.at[pl.ds(eoff, CHUNK)], idx_d_slots[b], sem_i)

    def idx_wait(i, b):
        eoff = ebase + i * CHUNK
        pltpu.make_async_copy(src.at[pl.ds(eoff, CHUNK)], idx_s_slots[b],
                              sem_i).wait()
        pltpu.make_async_copy(dst.at[pl.ds(eoff, CHUNK)], idx_d_slots[b],
                              sem_i).wait()

    idx_start(0, 0)
    idx_start(1, 1)
    idx_wait(0, 0)
    pltpu.async_copy(x2.at[idx_s0], rows0, sem_g)

    def body(i2, carry):
        for b in range(2):
            i = 2 * i2 + b
            nb = 1 - b

            @pl.when(i < NCHUNK - 1)
            def _():
                idx_wait(i + 1, nb)
                pltpu.async_copy(x2.at[idx_s_slots[nb]], rows_slots[nb],
                                 sem_g)

            pltpu.make_async_copy(x2.at[idx_s_slots[b]], rows_slots[b],
                                  sem_g).wait()
            pltpu.sync_copy(rows_slots[b], acc_sh.at[idx_d_slots[b]],
                            add=True)

            @pl.when(i < NCHUNK - 2)
            def _():
                idx_start(i + 2, b)
        return carry

    lax.fori_loop(0, NCHUNK // 2, body, 0)
    plsc.subcore_barrier()
    _spmem_writeback(acc_sh, rows0, s_out, s, c)


@functools.partial(
    pl.kernel,
    out_type=jax.ShapeDtypeStruct((NCORE, N, D), jnp.float32),
    mesh=_sc_mesh,
    scratch_types=(
        pltpu.VMEM_SHARED((NA, D), jnp.float32),
        pltpu.VMEM((CHUNK,), jnp.int32),
        pltpu.VMEM((CHUNK,), jnp.int32),
        pltpu.VMEM((CHUNK, D), jnp.float32),
        pltpu.VMEM((CHUNK, D), jnp.float32),
        pltpu.SemaphoreType.DMA,
    ),
)
def _sc_segcount(dst, c_out, cnt_sh, idx_d0, idx_d1, buf, ones, sem):
    c = lax.axis_index("c")
    s = lax.axis_index("s")
    _fill_rows(buf, CHUNK, 0.0)
    _fill_rows(ones, CHUNK, 1.0)
    _spmem_init(cnt_sh, buf, s)
    plsc.subcore_barrier()
    ebase = c * EPAD + s * EPT

    idx_slots = (idx_d0, idx_d1)
    pltpu.async_copy(dst.at[pl.ds(ebase, CHUNK)], idx_d0, sem)

    def body(i2, carry):
        for b in range(2):
            i = 2 * i2 + b
            eoff = ebase + i * CHUNK

            @pl.when(i < NCHUNK - 1)
            def _():
                pltpu.async_copy(dst.at[pl.ds(eoff + CHUNK, CHUNK)],
                                 idx_slots[1 - b], sem)

            pltpu.make_async_copy(dst.at[pl.ds(eoff, CHUNK)], idx_slots[b],
                                  sem).wait()
            pltpu.sync_copy(ones, cnt_sh.at[idx_slots[b]], add=True)
        return carry

    lax.fori_loop(0, NCHUNK // 2, body, 0)
    plsc.subcore_barrier()
    _spmem_writeback(cnt_sh, buf, c_out, s, c)


@functools.partial(
    pl.kernel,
    out_type=(
        jax.ShapeDtypeStruct((BPAD, D), jnp.float32),
        jax.ShapeDtypeStruct((BPAD, D), jnp.float32),
    ),
    mesh=_sc_mesh,
    scratch_types=(
        pltpu.VMEM((128,), jnp.int32),
        pltpu.VMEM((128,), jnp.int32),
        pltpu.VMEM((128,), jnp.int32),
        pltpu.VMEM((128,), jnp.int32),
        pltpu.VMEM((128, D), jnp.float32),
        pltpu.VMEM((128, D), jnp.float32),
        pltpu.VMEM((128, D), jnp.float32),
        pltpu.VMEM((128, D), jnp.float32),
        pltpu.SemaphoreType.DMA,
        pltpu.SemaphoreType.DMA,
    ),
)
def _sc_pair_gather(p2, i0, i1, a_out, b_out,
                    idx_a0, idx_a1, idx_b0, idx_b1,
                    rows_a0, rows_a1, rows_b0, rows_b1, sem_i, sem_g):
    c = lax.axis_index("c")
    s = lax.axis_index("s")
    wid = s * NCORE + c
    idx_a = (idx_a0, idx_a1)
    idx_b = (idx_b0, idx_b1)
    rows_a = (rows_a0, rows_a1)
    rows_b = (rows_b0, rows_b1)

    def idx_start(ch, b):
        off = ch * 128
        pltpu.async_copy(i0.at[pl.ds(off, 128)], idx_a[b], sem_i)
        pltpu.async_copy(i1.at[pl.ds(off, 128)], idx_b[b], sem_i)

    def idx_wait(ch, b):
        off = ch * 128
        pltpu.make_async_copy(i0.at[pl.ds(off, 128)], idx_a[b], sem_i).wait()
        pltpu.make_async_copy(i1.at[pl.ds(off, 128)], idx_b[b], sem_i).wait()

    def drain_write(ch, b):
        off = ch * 128
        pltpu.make_async_copy(p2.at[idx_a[b]], rows_a[b], sem_g).wait()
        pltpu.sync_copy(rows_a[b], a_out.at[pl.ds(off, 128)])
        pltpu.make_async_copy(p2.at[idx_b[b]], rows_b[b], sem_g).wait()
        pltpu.sync_copy(rows_b[b], b_out.at[pl.ds(off, 128)])

    ch0 = wid

    @pl.when(ch0 < NCH_PRED)
    def _():
        idx_start(ch0, 0)

    for j in range(PRED_ITERS):
        b = j % 2
        nb = 1 - b
        ch = wid + 32 * j
        nxt = ch + 32

        @pl.when(ch < NCH_PRED)
        def _():
            idx_wait(ch, b)
            pltpu.async_copy(p2.at[idx_a[b]], rows_a[b], sem_g)
            pltpu.async_copy(p2.at[idx_b[b]], rows_b[b], sem_g)

        if j > 0:
            pch = ch - 32

            @pl.when(pch < NCH_PRED)
            def _():
                drain_write(pch, nb)

        @pl.when(nxt < NCH_PRED)
        def _():
            idx_start(nxt, nb)

    lch = wid + 32 * (PRED_ITERS - 1)

    @pl.when(lch < NCH_PRED)
    def _():
        drain_write(lch, (PRED_ITERS - 1) % 2)


RB = 2000
NRB = N // RB


def _tc_layer1_body(s_ref, c_ref, x_ref, wl_ref, b_ref, wr_ref, out_ref):
    agg = s_ref[0] / jnp.maximum(c_ref[0], 1.0)
    h = (jnp.dot(agg, wl_ref[0], preferred_element_type=jnp.float32)
         + b_ref[0, 0][None, :]
         + jnp.dot(x_ref[...], wr_ref[0], preferred_element_type=jnp.float32))
    out_ref[...] = jnp.maximum(h, 0.0)


def _tc_layer2_body(s_ref, c_ref, x_ref, wl_ref, b_ref, wr_ref, wp_ref,
                    out_ref):
    agg = s_ref[0] / jnp.maximum(c_ref[0], 1.0)
    h = (jnp.dot(agg, wl_ref[0], preferred_element_type=jnp.float32)
         + b_ref[0, 0][None, :]
         + jnp.dot(x_ref[...], wr_ref[0], preferred_element_type=jnp.float32))
    out_ref[...] = jnp.dot(h, wp_ref[0], preferred_element_type=jnp.float32)


PB = 2048


def _tc_pred_body(a_ref, b_ref, bp1_ref, w2_ref, bp2_ref, out_ref):
    x = jnp.maximum(a_ref[...] + b_ref[...] + bp1_ref[...], 0.0)
    z = jnp.sum(x * w2_ref[...], axis=1) + bp2_ref[0, 0]
    out_ref[...] = jax.nn.sigmoid(z)


def kernel(x_investor, x_fund, edge_index_invests, edge_index_rev,
           edge_label_index,
           W1_if_l, W1_if_r, b1_if, W1_fi_l, W1_fi_r, b1_fi,
           W2_if_l, W2_if_r, b2_if, W2_fi_l, W2_fi_r, b2_fi,
           Wp1, bp1, Wp2, bp2):
    f32 = jnp.float32

    src = jnp.concatenate([edge_index_invests[0], edge_index_rev[0] + N])
    dst = jnp.concatenate([edge_index_invests[1], edge_index_rev[1]])

    cnt = _sc_segcount(dst)

    x1cat = jnp.concatenate([x_investor, x_fund])
    s1 = _sc_segsum(x1cat, src, dst)

    w1l = jnp.stack([W1_if_l, W1_fi_l])
    w1r = jnp.stack([W1_if_r, W1_fi_r])
    bias1 = jnp.stack([b1_if, b1_fi]).reshape(NCORE, 1, D)
    h1 = pl.pallas_call(
        _tc_layer1_body,
        grid=(NCORE, NRB),
        in_specs=[
            pl.BlockSpec((1, RB, D), lambda d, r: (d, r, 0)),
            pl.BlockSpec((1, RB, D), lambda d, r: (d, r, 0)),
            pl.BlockSpec((RB, D), lambda d, r: ((1 - d) * NRB + r, 0)),
            pl.BlockSpec((1, D, D), lambda d, r: (d, 0, 0)),
            pl.BlockSpec((1, 1, D), lambda d, r: (d, 0, 0)),
            pl.BlockSpec((1, D, D), lambda d, r: (d, 0, 0)),
        ],
        out_specs=pl.BlockSpec((RB, D), lambda d, r: ((1 - d) * NRB + r, 0)),
        out_shape=jax.ShapeDtypeStruct((NCORE * N, D), f32),
    )(s1, cnt, x1cat, w1l, bias1, w1r)

    s2 = _sc_segsum(h1, src, dst)

    w2l = jnp.stack([W2_if_l, W2_fi_l])
    w2r = jnp.stack([W2_if_r, W2_fi_r])
    bias2 = jnp.stack([b2_if, b2_fi]).reshape(NCORE, 1, D)
    wp_halves = jnp.stack([Wp1[D:], Wp1[:D]])
    p2 = pl.pallas_call(
        _tc_layer2_body,
        grid=(NCORE, NRB),
        in_specs=[
            pl.BlockSpec((1, RB, D), lambda d, r: (d, r, 0)),
            pl.BlockSpec((1, RB, D), lambda d, r: (d, r, 0)),
            pl.BlockSpec((RB, D), lambda d, r: ((1 - d) * NRB + r, 0)),
            pl.BlockSpec((1, D, D), lambda d, r: (d, 0, 0)),
            pl.BlockSpec((1, 1, D), lambda d, r: (d, 0, 0)),
            pl.BlockSpec((1, D, D), lambda d, r: (d, 0, 0)),
            pl.BlockSpec((1, D, D), lambda d, r: (d, 0, 0)),
        ],
        out_specs=pl.BlockSpec((RB, D), lambda d, r: (d * NRB + r, 0)),
        out_shape=jax.ShapeDtypeStruct((NCORE * N, D), f32),
    )(s2, cnt, h1, w2l, bias2, w2r, wp_halves)

    pad = jnp.zeros((BPAD - B,), jnp.int32)
    i0 = jnp.concatenate([edge_label_index[0] + N, pad])
    i1 = jnp.concatenate([edge_label_index[1], pad])
    ag, bg = _sc_pair_gather(p2, i0, i1)

    bp1r = bp1.reshape(1, D)
    w2row = Wp2.reshape(1, D)
    bp2r = jnp.broadcast_to(bp2.reshape(1, 1), (1, D))
    z = pl.pallas_call(
        _tc_pred_body,
        grid=(BPAD // PB,),
        in_specs=[
            pl.BlockSpec((PB, D), lambda r: (r, 0)),
            pl.BlockSpec((PB, D), lambda r: (r, 0)),
            pl.BlockSpec((1, D), lambda r: (0, 0)),
            pl.BlockSpec((1, D), lambda r: (0, 0)),
            pl.BlockSpec((1, D), lambda r: (0, 0)),
        ],
        out_specs=pl.BlockSpec((PB,), lambda r: (r,)),
        out_shape=jax.ShapeDtypeStruct((BPAD,), f32),
    )(ag, bg, bp1r, w2row, bp2r)
    return z[:B].reshape(B, 1)
